# Initial kernel scaffold; baseline (speedup 1.0000x reference)
import jax
import jax.numpy as jnp
from jax.experimental import pallas as pl


def kernel(x, gate_w, gate_proj_w, up_proj_w, down_proj_w):
    raise NotImplementedError("write your pallas kernel here")


# speedup vs baseline: 1.1787x; 1.1787x over previous
import functools

import jax
import jax.numpy as jnp
from jax import lax
from jax.experimental import pallas as pl
from jax.experimental.pallas import tpu as pltpu
from jax.experimental.pallas import tpu_sc as plsc

_B, _S, _H, _I, _E, _K = 2, 2048, 2048, 2048, 8, 2
_T = _B * _S
_TK = _T * _K

_TM = 256
_P = _TK + _E * _TM
_NT = _P // _TM
_IB = 512
_NJ = _I // _IB

_NC, _NS = 2, 16
_NW = _NC * _NS

_GC = 32
_CC = 16

_RT = 512


def _router_body(x_ref, g_ref, logits_ref, psum_ref):
    i = pl.program_id(0)
    logits = lax.dot_general(
        x_ref[...], g_ref[...], (((1,), (1,)), ((), ())),
        preferred_element_type=jnp.float32)
    logits_ref[...] = logits
    m = jnp.max(logits, axis=1, keepdims=True)
    p = jnp.exp(logits - m)
    p = p / jnp.sum(p, axis=1, keepdims=True)
    ps = jnp.sum(p, axis=0, keepdims=True)

    @pl.when(i == 0)
    def _():
        psum_ref[...] = ps

    @pl.when(i > 0)
    def _():
        psum_ref[...] += ps


def _router(x_flat, gate_w):
    return pl.pallas_call(
        _router_body,
        grid=(_T // _RT,),
        in_specs=[
            pl.BlockSpec((_RT, _H), lambda i: (i, 0)),
            pl.BlockSpec((_E, _H), lambda i: (0, 0)),
        ],
        out_specs=[
            pl.BlockSpec((_RT, _E), lambda i: (i, 0)),
            pl.BlockSpec((1, _E), lambda i: (0, 0)),
        ],
        out_shape=[
            jax.ShapeDtypeStruct((_T, _E), jnp.float32),
            jax.ShapeDtypeStruct((1, _E), jnp.float32),
        ],
    )(x_flat, gate_w)


@functools.lru_cache(maxsize=None)
def _make_sc_gather():
    mesh = plsc.VectorSubcoreMesh(core_axis_name="c", subcore_axis_name="s", num_cores=_NC, num_subcores=_NS)
    rows_per_w = _P // _NW

    @functools.partial(
        pl.kernel,
        out_type=jax.ShapeDtypeStruct((_P, _H), jnp.float32),
        mesh=mesh,
        scratch_types=[
            pltpu.VMEM((_GC,), jnp.int32),
            pltpu.VMEM((_GC, _H), jnp.float32),
            pltpu.SemaphoreType.DMA,
        ],
    )
    def gather(x_hbm, idx_hbm, out_hbm, idx_v, rows_v, sem):
        wid = lax.axis_index("s") * _NC + lax.axis_index("c")
        base = wid * rows_per_w

        def body(it, carry):
            off = base + it * _GC
            pltpu.sync_copy(idx_hbm.at[pl.ds(off, _GC)], idx_v)
            pltpu.async_copy(x_hbm.at[idx_v], rows_v, sem).wait()
            pltpu.sync_copy(rows_v, out_hbm.at[pl.ds(off, _GC)])
            return carry

        lax.fori_loop(0, rows_per_w // _GC, body, 0)

    return gather


def _ffn_body(eot_ref, xs_ref, gw_ref, uw_ref, dw_ref, w_ref, out_ref):
    del eot_ref
    j = pl.program_id(1)
    xs = xs_ref[...]
    g = lax.dot_general(xs, gw_ref[0], (((1,), (1,)), ((), ())),
                        preferred_element_type=jnp.float32)
    u = lax.dot_general(xs, uw_ref[0], (((1,), (1,)), ((), ())),
                        preferred_element_type=jnp.float32)
    h = (g * u) / (1.0 + jnp.exp(-g))
    contrib = lax.dot_general(h, dw_ref[0], (((1,), (1,)), ((), ())),
                              preferred_element_type=jnp.float32)

    @pl.when(j == 0)
    def _():
        out_ref[...] = contrib

    @pl.when(j > 0)
    def _():
        out_ref[...] += contrib

    @pl.when(j == _NJ - 1)
    def _():
        w = w_ref[0, 0, :]
        out_ref[...] *= w[:, None]


def _grouped_ffn(eot, xs, gate_proj_w, up_proj_w, down_proj_w, w_tiles):
    grid_spec = pltpu.PrefetchScalarGridSpec(
        num_scalar_prefetch=1,
        grid=(_NT, _NJ),
        in_specs=[
            pl.BlockSpec((_TM, _H), lambda i, j, eot: (i, 0)),
            pl.BlockSpec((1, _IB, _H), lambda i, j, eot: (eot[i], j, 0)),
            pl.BlockSpec((1, _IB, _H), lambda i, j, eot: (eot[i], j, 0)),
            pl.BlockSpec((1, _H, _IB), lambda i, j, eot: (eot[i], 0, j)),
            pl.BlockSpec((1, 1, _TM), lambda i, j, eot: (i, 0, 0)),
        ],
        out_specs=pl.BlockSpec((_TM, _H), lambda i, j, eot: (i, 0)),
    )
    return pl.pallas_call(
        _ffn_body,
        grid_spec=grid_spec,
        out_shape=jax.ShapeDtypeStruct((_P, _H), jnp.float32),
        compiler_params=pltpu.CompilerParams(
            dimension_semantics=("arbitrary", "arbitrary")),
    )(eot, xs, gate_proj_w, up_proj_w, down_proj_w, w_tiles)


@functools.lru_cache(maxsize=None)
def _make_sc_combine():
    mesh = plsc.VectorSubcoreMesh(core_axis_name="c", subcore_axis_name="s", num_cores=_NC, num_subcores=_NS)
    tok_per_w = _T // _NW

    @functools.partial(
        pl.kernel,
        out_type=jax.ShapeDtypeStruct((_T, _H), jnp.float32),
        mesh=mesh,
        scratch_types=[
            pltpu.VMEM((_CC,), jnp.int32),
            pltpu.VMEM((_CC,), jnp.int32),
            pltpu.VMEM((_CC, _H), jnp.float32),
            pltpu.VMEM((_CC, _H), jnp.float32),
            pltpu.SemaphoreType.DMA,
            pltpu.SemaphoreType.DMA,
        ],
    )
    def combine(ys_hbm, p0_hbm, p1_hbm, out_hbm, i0_v, i1_v, a_v, b_v,
                sem0, sem1):
        wid = lax.axis_index("s") * _NC + lax.axis_index("c")
        base = wid * tok_per_w

        def body(it, carry):
            off = base + it * _CC
            pltpu.sync_copy(p0_hbm.at[pl.ds(off, _CC)], i0_v)
            pltpu.sync_copy(p1_hbm.at[pl.ds(off, _CC)], i1_v)
            cp0 = pltpu.async_copy(ys_hbm.at[i0_v], a_v, sem0)
            cp1 = pltpu.async_copy(ys_hbm.at[i1_v], b_v, sem1)
            cp0.wait()
            cp1.wait()

            def tok(t, c):
                def vec(v, c2):
                    sl = pl.ds(v * 16, 16)
                    a_v[t, sl] = a_v[t, sl] + b_v[t, sl]
                    return c2
                return lax.fori_loop(0, _H // 16, vec, c, unroll=8)

            lax.fori_loop(0, _CC, tok, 0)
            pltpu.sync_copy(a_v, out_hbm.at[pl.ds(off, _CC)])
            return carry

        lax.fori_loop(0, tok_per_w // _CC, body, 0)

    return combine


def kernel(x, gate_w, gate_proj_w, up_proj_w, down_proj_w):
    x_flat = x.reshape(_T, _H)

    router_logits, psum = _router(x_flat, gate_w)

    top_vals, selected = lax.top_k(router_logits, _K)
    routing_weights = jax.nn.softmax(top_vals, axis=-1)

    flat_e = selected.reshape(-1).astype(jnp.int32)
    order = jnp.argsort(flat_e, stable=True)
    sorted_e = flat_e[order]
    counts = jnp.bincount(flat_e, length=_E).astype(jnp.int32)
    group_start = jnp.concatenate(
        [jnp.zeros(1, jnp.int32), jnp.cumsum(counts)[:-1].astype(jnp.int32)])
    caps = ((counts + _TM - 1) // _TM) * _TM
    ends = jnp.cumsum(caps).astype(jnp.int32)
    pad_off = ends - caps

    jj = jnp.arange(_TK, dtype=jnp.int32)
    pp = pad_off[sorted_e] + (jj - group_start[sorted_e])

    src_token = jnp.zeros(_P, jnp.int32).at[pp].set(
        (order // _K).astype(jnp.int32))
    w_sorted = jnp.zeros(_P, jnp.float32).at[pp].set(
        routing_weights.reshape(-1)[order])
    posf = jnp.zeros(_TK, jnp.int32).at[order].set(pp)
    pos0 = posf.reshape(_T, _K)[:, 0]
    pos1 = posf.reshape(_T, _K)[:, 1]

    tile_start = jnp.arange(_NT, dtype=jnp.int32) * _TM
    eot = jnp.minimum(
        jnp.searchsorted(ends, tile_start, side="right"), _E - 1
    ).astype(jnp.int32)

    xs = _make_sc_gather()(x_flat, src_token)

    ys = _grouped_ffn(eot, xs, gate_proj_w, up_proj_w, down_proj_w,
                      w_sorted.reshape(_NT, 1, _TM))

    final = _make_sc_combine()(ys, pos0, pos1)

    counts_f = counts.astype(jnp.float32)
    aux_loss = jnp.sum((psum[0] / _T) * (counts_f / _T))

    return final.reshape(x.shape), aux_loss


# speedup vs baseline: 1.3094x; 1.1109x over previous
import functools

import jax
import jax.numpy as jnp
from jax import lax
from jax.experimental import pallas as pl
from jax.experimental.pallas import tpu as pltpu
from jax.experimental.pallas import tpu_sc as plsc

_B, _S, _H, _I, _E, _K = 2, 2048, 2048, 2048, 8, 2
_T = _B * _S
_TK = _T * _K

_TM = 256
_P = _TK + _E * _TM
_NT = _P // _TM

_NC, _NS = 2, 16
_NW = _NC * _NS

_GC = 16
_GN = (_P // _NW) // _GC
_CC = 8
_CN = (_T // _NW) // _CC

_RT = 512


def _router_body(x_ref, g_ref, logits_ref, psum_ref):
    i = pl.program_id(0)
    logits = lax.dot_general(
        x_ref[...], g_ref[...], (((1,), (1,)), ((), ())),
        preferred_element_type=jnp.float32)
    logits_ref[...] = logits
    m = jnp.max(logits, axis=1, keepdims=True)
    p = jnp.exp(logits - m)
    p = p / jnp.sum(p, axis=1, keepdims=True)
    ps = jnp.sum(p, axis=0, keepdims=True)

    @pl.when(i == 0)
    def _():
        psum_ref[...] = ps

    @pl.when(i > 0)
    def _():
        psum_ref[...] += ps


def _router(x_flat, gate_w):
    return pl.pallas_call(
        _router_body,
        grid=(_T // _RT,),
        in_specs=[
            pl.BlockSpec((_RT, _H), lambda i: (i, 0)),
            pl.BlockSpec((_E, _H), lambda i: (0, 0)),
        ],
        out_specs=[
            pl.BlockSpec((_RT, _E), lambda i: (i, 0)),
            pl.BlockSpec((1, _E), lambda i: (0, 0)),
        ],
        out_shape=[
            jax.ShapeDtypeStruct((_T, _E), jnp.float32),
            jax.ShapeDtypeStruct((1, _E), jnp.float32),
        ],
    )(x_flat, gate_w)


@functools.lru_cache(maxsize=None)
def _make_sc_gather():
    mesh = plsc.VectorSubcoreMesh(
        core_axis_name="c", subcore_axis_name="s",
        num_cores=_NC, num_subcores=_NS)
    rows_per_w = _P // _NW

    @functools.partial(
        pl.kernel,
        out_type=jax.ShapeDtypeStruct((_P, _H), jnp.float32),
        mesh=mesh,
        scratch_types=[
            pltpu.VMEM((_GN, _GC), jnp.int32),
            pltpu.VMEM((2, _GC, _H), jnp.float32),
            pltpu.SemaphoreType.DMA,
            pltpu.SemaphoreType.DMA,
            pltpu.SemaphoreType.DMA,
            pltpu.SemaphoreType.DMA,
        ],
    )
    def gather(x_hbm, idx_hbm, out_hbm, idx_v, rows_v, sg0, sg1, sw0, sw1):
        wid = lax.axis_index("s") * _NC + lax.axis_index("c")
        base = wid * rows_per_w
        pltpu.sync_copy(idx_hbm.at[wid], idx_v)
        sg = (sg0, sg1)
        sw = (sw0, sw1)

        def group(g, carry):
            it0 = g * 2
            cps = []
            for b in range(2):
                cps.append(pltpu.async_copy(
                    x_hbm.at[idx_v.at[it0 + b]], rows_v.at[b], sg[b]))
            wbs = []
            for b in range(2):
                cps[b].wait()
                off = base + (it0 + b) * _GC
                wbs.append(pltpu.async_copy(
                    rows_v.at[b], out_hbm.at[pl.ds(off, _GC)], sw[b]))
            for b in range(2):
                wbs[b].wait()
            return carry

        lax.fori_loop(0, _GN // 2, group, 0)

    return gather


def _ffn_body(eot_ref, xs_ref, gw_ref, uw_ref, dw_ref, w_ref, out_ref):
    del eot_ref
    xs = xs_ref[...].astype(jnp.bfloat16)
    g = lax.dot_general(xs, gw_ref[0], (((1,), (1,)), ((), ())),
                        preferred_element_type=jnp.float32)
    u = lax.dot_general(xs, uw_ref[0], (((1,), (1,)), ((), ())),
                        preferred_element_type=jnp.float32)
    h = ((g * u) / (1.0 + jnp.exp(-g))).astype(jnp.bfloat16)
    contrib = lax.dot_general(h, dw_ref[0], (((1,), (1,)), ((), ())),
                              preferred_element_type=jnp.float32)
    w = w_ref[0, 0, :]
    out_ref[...] = contrib * w[:, None]


def _grouped_ffn(eot, xs, gate_proj_w, up_proj_w, down_proj_w, w_tiles):
    grid_spec = pltpu.PrefetchScalarGridSpec(
        num_scalar_prefetch=1,
        grid=(_NT,),
        in_specs=[
            pl.BlockSpec((_TM, _H), lambda i, eot: (i, 0)),
            pl.BlockSpec((1, _I, _H), lambda i, eot: (eot[i], 0, 0)),
            pl.BlockSpec((1, _I, _H), lambda i, eot: (eot[i], 0, 0)),
            pl.BlockSpec((1, _H, _I), lambda i, eot: (eot[i], 0, 0)),
            pl.BlockSpec((1, 1, _TM), lambda i, eot: (i, 0, 0)),
        ],
        out_specs=pl.BlockSpec((_TM, _H), lambda i, eot: (i, 0)),
    )
    return pl.pallas_call(
        _ffn_body,
        grid_spec=grid_spec,
        out_shape=jax.ShapeDtypeStruct((_P, _H), jnp.float32),
        compiler_params=pltpu.CompilerParams(
            dimension_semantics=("arbitrary",),
            vmem_limit_bytes=100 * 1024 * 1024),
    )(eot, xs, gate_proj_w, up_proj_w, down_proj_w, w_tiles)


@functools.lru_cache(maxsize=None)
def _make_sc_combine():
    mesh = plsc.VectorSubcoreMesh(
        core_axis_name="c", subcore_axis_name="s",
        num_cores=_NC, num_subcores=_NS)
    tok_per_w = _T // _NW

    @functools.partial(
        pl.kernel,
        out_type=jax.ShapeDtypeStruct((_T, _H), jnp.float32),
        mesh=mesh,
        scratch_types=[
            pltpu.VMEM((_CN, _CC), jnp.int32),
            pltpu.VMEM((_CN, _CC), jnp.int32),
            pltpu.VMEM((2, _CC, _H), jnp.float32),
            pltpu.VMEM((2, _CC, _H), jnp.float32),
            pltpu.SemaphoreType.DMA,
            pltpu.SemaphoreType.DMA,
            pltpu.SemaphoreType.DMA,
            pltpu.SemaphoreType.DMA,
            pltpu.SemaphoreType.DMA,
            pltpu.SemaphoreType.DMA,
        ],
    )
    def combine(ys_hbm, p0_hbm, p1_hbm, out_hbm, i0_v, i1_v, a_v, b_v,
                sa0, sa1, sb0, sb1, sw0, sw1):
        wid = lax.axis_index("s") * _NC + lax.axis_index("c")
        base = wid * tok_per_w
        pltpu.sync_copy(p0_hbm.at[wid], i0_v)
        pltpu.sync_copy(p1_hbm.at[wid], i1_v)
        sa = (sa0, sa1)
        sb = (sb0, sb1)
        sw = (sw0, sw1)

        def group(g, carry):
            it0 = g * 2
            cps = []
            for b in range(2):
                cps.append((
                    pltpu.async_copy(
                        ys_hbm.at[i0_v.at[it0 + b]], a_v.at[b], sa[b]),
                    pltpu.async_copy(
                        ys_hbm.at[i1_v.at[it0 + b]], b_v.at[b], sb[b]),
                ))
            wbs = []
            for b in range(2):
                cps[b][0].wait()
                cps[b][1].wait()

                def tok(t, c, b=b):
                    def vec(v, c2, b=b):
                        sl = pl.ds(v * 16, 16)
                        a_v[b, t, sl] = a_v[b, t, sl] + b_v[b, t, sl]
                        return c2
                    return lax.fori_loop(0, _H // 16, vec, c, unroll=8)

                lax.fori_loop(0, _CC, tok, 0)
                off = base + (it0 + b) * _CC
                wbs.append(pltpu.async_copy(
                    a_v.at[b], out_hbm.at[pl.ds(off, _CC)], sw[b]))
            for b in range(2):
                wbs[b].wait()
            return carry

        lax.fori_loop(0, _CN // 2, group, 0)

    return combine


def kernel(x, gate_w, gate_proj_w, up_proj_w, down_proj_w):
    x_flat = x.reshape(_T, _H)

    router_logits, psum = _router(x_flat, gate_w)

    top_vals, selected = lax.top_k(router_logits, _K)
    routing_weights = jax.nn.softmax(top_vals, axis=-1)

    flat_e = selected.reshape(-1).astype(jnp.int32)
    order = jnp.argsort(flat_e, stable=True)
    sorted_e = flat_e[order]
    counts = jnp.bincount(flat_e, length=_E).astype(jnp.int32)
    group_start = jnp.concatenate(
        [jnp.zeros(1, jnp.int32), jnp.cumsum(counts)[:-1].astype(jnp.int32)])
    caps = ((counts + _TM - 1) // _TM) * _TM
    ends = jnp.cumsum(caps).astype(jnp.int32)
    pad_off = ends - caps

    jj = jnp.arange(_TK, dtype=jnp.int32)
    pp = pad_off[sorted_e] + (jj - group_start[sorted_e])

    src_token = jnp.zeros(_P, jnp.int32).at[pp].set(
        (order // _K).astype(jnp.int32))
    w_sorted = jnp.zeros(_P, jnp.float32).at[pp].set(
        routing_weights.reshape(-1)[order])
    posf = jnp.zeros(_TK, jnp.int32).at[order].set(pp)
    pos0 = posf.reshape(_T, _K)[:, 0]
    pos1 = posf.reshape(_T, _K)[:, 1]

    tile_start = jnp.arange(_NT, dtype=jnp.int32) * _TM
    eot = jnp.minimum(
        jnp.searchsorted(ends, tile_start, side="right"), _E - 1
    ).astype(jnp.int32)

    xs = _make_sc_gather()(
        x_flat, src_token.reshape(_NW, _GN, _GC))

    ys = _grouped_ffn(eot, xs,
                      gate_proj_w.astype(jnp.bfloat16),
                      up_proj_w.astype(jnp.bfloat16),
                      down_proj_w.astype(jnp.bfloat16),
                      w_sorted.reshape(_NT, 1, _TM))

    final = _make_sc_combine()(
        ys,
        pos0.reshape(_NW, _CN, _CC),
        pos1.reshape(_NW, _CN, _CC))

    counts_f = counts.astype(jnp.float32)
    aux_loss = jnp.sum((psum[0] / _T) * (counts_f / _T))

    return final.reshape(x.shape), aux_loss
